# baseline (speedup 1.0000x reference)
import functools

import jax
import jax.numpy as jnp
from jax import lax
from jax.experimental import pallas as pl
from jax.experimental.pallas import tpu as pltpu
from jax.experimental.pallas import tpu_sc as plsc

NC = 2
NS = 16
NW = NC * NS
K = 16


@functools.lru_cache(maxsize=None)
def _make_sc_gather(B, D):
  assert B % (8 * NW) == 0
  b_per_w = B // NW
  assert b_per_w % K == 0
  n_grp = b_per_w // K
  mesh = plsc.VectorSubcoreMesh(core_axis_name="c", subcore_axis_name="s")

  @functools.partial(
      pl.kernel,
      out_type=jax.ShapeDtypeStruct((B, D), jnp.float32),
      mesh=mesh,
      scratch_types=[
          pltpu.VMEM((b_per_w,), jnp.int32),
          pltpu.VMEM((b_per_w, D), jnp.float32),
          pltpu.SemaphoreType.DMA,
      ],
  )
  def gather_kernel(idx_hbm, tab_hbm, out_hbm, idx_v, rows_v, sem):
    wid = lax.axis_index("s") * NC + lax.axis_index("c")
    base = wid * b_per_w
    pltpu.sync_copy(idx_hbm.at[pl.ds(base, b_per_w)], idx_v)

    def issue_group(g):
      v = idx_v[pl.ds(g * K, K)]
      for k in range(K):
        pltpu.async_copy(tab_hbm.at[pl.ds(v[k], 1)],
                         rows_v.at[pl.ds(g * K + k, 1)], sem)

    def drain_group():
      for _ in range(K):
        pltpu.make_async_copy(tab_hbm.at[pl.ds(0, 1)],
                              rows_v.at[pl.ds(0, 1)], sem).wait()

    def body(g, carry):
      issue_group(g)

      @pl.when(g > 0)
      def _():
        drain_group()

      return carry

    lax.fori_loop(0, n_grp, body, 0)
    drain_group()
    pltpu.sync_copy(rows_v, out_hbm.at[pl.ds(base, b_per_w)])

  return gather_kernel


CH = 128


@functools.lru_cache(maxsize=None)
def _make_sc_gather_linear(B, D):
  assert B % (8 * NW) == 0
  b_per_w = B // NW
  assert b_per_w % CH == 0
  n_ch = b_per_w // CH
  mesh = plsc.VectorSubcoreMesh(core_axis_name="c", subcore_axis_name="s")

  @functools.partial(
      pl.kernel,
      out_type=jax.ShapeDtypeStruct((B, D), jnp.float32),
      mesh=mesh,
      compiler_params=pltpu.CompilerParams(use_tc_tiling_on_sc=False),
      scratch_types=[
          pltpu.VMEM((b_per_w,), jnp.int32),
          pltpu.VMEM((b_per_w, D), jnp.float32),
          pltpu.SemaphoreType.DMA,
      ],
  )
  def gather_kernel(idx_hbm, tab_hbm, out_hbm, idx_v, rows_v, sem):
    wid = lax.axis_index("s") * NC + lax.axis_index("c")
    base = wid * b_per_w
    pltpu.sync_copy(idx_hbm.at[pl.ds(base, b_per_w)], idx_v)
    copies = []
    for c in range(n_ch):
      sl = pl.ds(c * CH, CH)
      copies.append(pltpu.async_copy(tab_hbm.at[idx_v.at[sl]], rows_v.at[sl], sem))
    for cp in copies:
      cp.wait()
    pltpu.sync_copy(rows_v, out_hbm.at[pl.ds(base, b_per_w)])

  return gather_kernel


def _tc_body(u_ref, i_ref, a1, c1, a2, c2, a3, c3, a4, c4, o_ref):
  f32 = jnp.float32
  x = u_ref[...]
  h = jnp.maximum(jnp.dot(x, a1[...], preferred_element_type=f32) + c1[...], 0.0)
  h = jnp.maximum(jnp.dot(h, a2[...], preferred_element_type=f32) + c2[...], 0.0)
  h = jnp.maximum(jnp.dot(h, a3[...], preferred_element_type=f32) + c3[...], 0.0)
  o = jnp.dot(h, a4[...], preferred_element_type=f32) + c4[...]
  o_ref[...] = jnp.sum(o * i_ref[...], axis=1, keepdims=True) * (1.0 / 3.0)


@jax.jit
def kernel(user, item, su_table, ti_table, mlp1, mlp2, mlp3):
  B = user.shape[0]
  D = su_table.shape[1]
  uidx = user.astype(jnp.int32)
  iidx = item.astype(jnp.int32)

  u_emb = _make_sc_gather(B, D)(uidx, su_table)
  i_emb = _make_sc_gather_linear(B, D)(iidx, ti_table)

  mlps = (mlp1, mlp2, mlp3)
  a1 = jnp.concatenate([m[0][0] for m in mlps], axis=1)
  c1 = jnp.concatenate([m[0][1] for m in mlps])
  H = mlp1[0][0].shape[1]

  def blockdiag(layer):
    z = jnp.zeros((3 * H, 3 * H), jnp.float32)
    for k, m in enumerate(mlps):
      z = z.at[k * H:(k + 1) * H, k * H:(k + 1) * H].set(m[layer][0])
    return z

  a2 = blockdiag(1)
  c2 = jnp.concatenate([m[1][1] for m in mlps])
  a3 = blockdiag(2)
  c3 = jnp.concatenate([m[2][1] for m in mlps])
  a4 = jnp.concatenate([m[3][0] for m in mlps], axis=0)
  c4 = mlp1[3][1] + mlp2[3][1] + mlp3[3][1]

  BLK = 4096
  row_blk = lambda w: pl.BlockSpec((BLK, w), lambda i: (i, 0))
  full = lambda r, c: pl.BlockSpec((r, c), lambda i: (0, 0))
  score = pl.pallas_call(
      _tc_body,
      grid=(B // BLK,),
      in_specs=[row_blk(D), row_blk(D),
                full(D, 3 * H), full(1, 3 * H), full(3 * H, 3 * H),
                full(1, 3 * H), full(3 * H, 3 * H), full(1, 3 * H),
                full(3 * H, D), full(1, D)],
      out_specs=row_blk(1),
      out_shape=jax.ShapeDtypeStruct((B, 1), jnp.float32),
  )(u_emb, i_emb,
    a1, c1.reshape(1, -1), a2, c2.reshape(1, -1),
    a3, c3.reshape(1, -1), a4, c4.reshape(1, -1))
  return score.reshape(B)


# speedup vs baseline: 1.3055x; 1.3055x over previous
import functools

import jax
import jax.numpy as jnp
from jax import lax
from jax.experimental import pallas as pl
from jax.experimental.pallas import tpu as pltpu
from jax.experimental.pallas import tpu_sc as plsc

NC = 2
NS = 16
NW = NC * NS
K = 16


@functools.lru_cache(maxsize=None)
def _make_sc_gather(B, D):
  assert B % (8 * NW) == 0
  b_per_w = B // NW
  assert b_per_w % K == 0
  n_grp = b_per_w // K
  mesh = plsc.VectorSubcoreMesh(core_axis_name="c", subcore_axis_name="s")

  @functools.partial(
      pl.kernel,
      out_type=(
          jax.ShapeDtypeStruct((B, D), jnp.float32),
          jax.ShapeDtypeStruct((B, D), jnp.float32),
      ),
      mesh=mesh,
      scratch_types=[
          pltpu.VMEM((b_per_w,), jnp.int32),
          pltpu.VMEM((b_per_w, D), jnp.float32),
          pltpu.SemaphoreType.DMA,
      ],
  )
  def gather_kernel(uidx_hbm, iidx_hbm, su_hbm, ti_hbm, uo_hbm, io_hbm,
                    idx_v, rows_v, sem):
    wid = lax.axis_index("s") * NC + lax.axis_index("c")
    base = wid * b_per_w

    def one_table(idx_hbm, tab_hbm, out_hbm):
      pltpu.sync_copy(idx_hbm.at[pl.ds(base, b_per_w)], idx_v)

      def issue_group(g):
        v = idx_v[pl.ds(g * K, K)]
        for k in range(K):
          pltpu.async_copy(tab_hbm.at[pl.ds(v[k], 1)],
                           rows_v.at[pl.ds(g * K + k, 1)], sem)

      def drain_group():
        for _ in range(K):
          pltpu.make_async_copy(tab_hbm.at[pl.ds(0, 1)],
                                rows_v.at[pl.ds(0, 1)], sem).wait()

      def body(g, carry):
        issue_group(g)

        @pl.when(g > 0)
        def _():
          drain_group()

        return carry

      lax.fori_loop(0, n_grp, body, 0)
      drain_group()
      pltpu.sync_copy(rows_v, out_hbm.at[pl.ds(base, b_per_w)])

    one_table(uidx_hbm, su_hbm, uo_hbm)
    one_table(iidx_hbm, ti_hbm, io_hbm)

  return gather_kernel


def _tc_body(u_ref, i_ref, a1, c1, a2, c2, a3, c3, a4, c4, o_ref):
  f32 = jnp.float32
  x = u_ref[...]
  h = jnp.maximum(jnp.dot(x, a1[...], preferred_element_type=f32) + c1[...], 0.0)
  h = jnp.maximum(jnp.dot(h, a2[...], preferred_element_type=f32) + c2[...], 0.0)
  h = jnp.maximum(jnp.dot(h, a3[...], preferred_element_type=f32) + c3[...], 0.0)
  o = jnp.dot(h, a4[...], preferred_element_type=f32) + c4[...]
  o_ref[...] = jnp.sum(o * i_ref[...], axis=1, keepdims=True) * (1.0 / 3.0)


@jax.jit
def kernel(user, item, su_table, ti_table, mlp1, mlp2, mlp3):
  B = user.shape[0]
  D = su_table.shape[1]
  uidx = user.astype(jnp.int32)
  iidx = item.astype(jnp.int32)

  u_emb, i_emb = _make_sc_gather(B, D)(uidx, iidx, su_table, ti_table)

  mlps = (mlp1, mlp2, mlp3)
  a1 = jnp.concatenate([m[0][0] for m in mlps], axis=1)
  c1 = jnp.concatenate([m[0][1] for m in mlps])
  H = mlp1[0][0].shape[1]

  def blockdiag(layer):
    z = jnp.zeros((3 * H, 3 * H), jnp.float32)
    for k, m in enumerate(mlps):
      z = z.at[k * H:(k + 1) * H, k * H:(k + 1) * H].set(m[layer][0])
    return z

  a2 = blockdiag(1)
  c2 = jnp.concatenate([m[1][1] for m in mlps])
  a3 = blockdiag(2)
  c3 = jnp.concatenate([m[2][1] for m in mlps])
  a4 = jnp.concatenate([m[3][0] for m in mlps], axis=0)
  c4 = mlp1[3][1] + mlp2[3][1] + mlp3[3][1]

  BLK = 4096
  row_blk = lambda w: pl.BlockSpec((BLK, w), lambda i: (i, 0))
  full = lambda r, c: pl.BlockSpec((r, c), lambda i: (0, 0))
  score = pl.pallas_call(
      _tc_body,
      grid=(B // BLK,),
      in_specs=[row_blk(D), row_blk(D),
                full(D, 3 * H), full(1, 3 * H), full(3 * H, 3 * H),
                full(1, 3 * H), full(3 * H, 3 * H), full(1, 3 * H),
                full(3 * H, D), full(1, D)],
      out_specs=row_blk(1),
      out_shape=jax.ShapeDtypeStruct((B, 1), jnp.float32),
  )(u_emb, i_emb,
    a1, c1.reshape(1, -1), a2, c2.reshape(1, -1),
    a3, c3.reshape(1, -1), a4, c4.reshape(1, -1))
  return score.reshape(B)


# speedup vs baseline: 1.4580x; 1.1167x over previous
import functools

import jax
import jax.numpy as jnp
from jax import lax
from jax.experimental import pallas as pl
from jax.experimental.pallas import tpu as pltpu
from jax.experimental.pallas import tpu_sc as plsc

NC = 2
NS = 16
NW = NC * NS
CH = 128
PACK = 4
WIN = 512
NWIN = 31
PBLK = WIN * NWIN


def _pack_body(t_ref, o_ref):
  x = t_ref[...]
  parts = []
  for w in range(NWIN):
    band = [jnp.transpose(x[:, WIN * w + 128 * a:WIN * w + 128 * (a + 1)])
            for a in range(PACK)]
    parts.append(jnp.concatenate(band, axis=1))
  o_ref[...] = jnp.concatenate(parts, axis=0)


def _pack_tail_body(m_ref, t_ref, o_ref):
  del m_ref
  tr = o_ref.shape[0]
  o_ref[...] = jnp.concatenate(
      [t_ref[a * tr:(a + 1) * tr, :] for a in range(PACK)], axis=1)


@functools.lru_cache(maxsize=None)
def _make_pack(V, D):
  n_main = V // PBLK
  main_cols = n_main * PBLK
  tail_cols = V - main_cols
  assert tail_cols % PACK == 0
  W = PACK * D
  main = pl.pallas_call(
      _pack_body,
      grid=(n_main,),
      in_specs=[pl.BlockSpec((D, PBLK), lambda g: (0, g))],
      out_specs=pl.BlockSpec((PBLK // PACK, W), lambda g: (g, 0)),
      out_shape=jax.ShapeDtypeStruct((V // PACK, W), jnp.float32),
  )
  if tail_cols == 0:
    return lambda tab, tab_t: main(tab_t)
  tail_rows = tail_cols // PACK
  assert tail_rows & (tail_rows - 1) == 0
  tail_blk_row = main_cols // PACK // tail_rows

  def pack(tab, tab_t):
    packed = main(tab_t)
    tab_tail = lax.slice(tab, (main_cols, 0), (V, D))
    return pl.pallas_call(
        _pack_tail_body,
        grid=(1,),
        in_specs=[pl.BlockSpec((tail_rows, W), lambda g: (tail_blk_row, 0)),
                  pl.BlockSpec((tail_cols, D), lambda g: (0, 0))],
        out_specs=pl.BlockSpec((tail_rows, W), lambda g: (tail_blk_row, 0)),
        out_shape=jax.ShapeDtypeStruct((V // PACK, W), jnp.float32),
        input_output_aliases={0: 0},
    )(packed, tab_tail)

  return pack


@functools.lru_cache(maxsize=None)
def _make_sc_gather(B, W, main_cols, tail_rows):
  assert B % (8 * NW) == 0
  b_per_w = B // NW
  assert b_per_w % CH == 0
  n_ch = b_per_w // CH
  main_rows = main_cols // PACK
  tail_shift = max(tail_rows.bit_length() - 1, 0)
  mesh = plsc.VectorSubcoreMesh(core_axis_name="c", subcore_axis_name="s")

  @functools.partial(
      pl.kernel,
      out_type=(
          jax.ShapeDtypeStruct((B, W), jnp.float32),
          jax.ShapeDtypeStruct((B, W), jnp.float32),
      ),
      mesh=mesh,
      scratch_types=[
          pltpu.VMEM((b_per_w,), jnp.int32),
          pltpu.VMEM((b_per_w,), jnp.int32),
          pltpu.VMEM((b_per_w, W), jnp.float32),
          pltpu.SemaphoreType.DMA,
      ],
  )
  def gather_kernel(uidx_hbm, iidx_hbm, su_hbm, ti_hbm, uo_hbm, io_hbm,
                    idx_v, idx4_v, rows_v, sem):
    wid = lax.axis_index("s") * NC + lax.axis_index("c")
    base = wid * b_per_w

    def one_table(idx_hbm, tab_hbm, out_hbm):
      pltpu.sync_copy(idx_hbm.at[pl.ds(base, b_per_w)], idx_v)
      for k in range(b_per_w // 16):
        sl = pl.ds(16 * k, 16)
        v = idx_v[sl]
        main_row = ((v >> 9) << 7) | (v & 127)
        tail_row = main_rows + ((v - main_cols) & (tail_rows - 1))
        idx4_v[sl] = jnp.where(v < main_cols, main_row, tail_row)
      copies = []
      for c in range(n_ch):
        sl = pl.ds(c * CH, CH)
        copies.append(pltpu.async_copy(tab_hbm.at[idx4_v.at[sl]], rows_v.at[sl], sem))
      for cp in copies:
        cp.wait()
      pltpu.sync_copy(rows_v, out_hbm.at[pl.ds(base, b_per_w)])

    one_table(uidx_hbm, su_hbm, uo_hbm)
    one_table(iidx_hbm, ti_hbm, io_hbm)

  return gather_kernel


def _tc_body(u4_ref, i4_ref, uq_ref, iq_ref, a1, c1, a2, c2, a3, c3, a4, c4,
             o_ref, *, main_cols, tail_shift):
  f32 = jnp.float32
  D = a1.shape[0]

  def band(idx):
    return jnp.where(idx < main_cols, (idx >> 7) & (PACK - 1),
                     (idx - main_cols) >> tail_shift)

  def select(g4, q):
    acc = jnp.where(q == 0, g4[:, 0:D], 0.0)
    for k in range(1, PACK):
      acc = acc + jnp.where(q == k, g4[:, k * D:(k + 1) * D], 0.0)
    return acc

  x = select(u4_ref[...], band(uq_ref[...]))
  iemb = select(i4_ref[...], band(iq_ref[...]))
  h = jnp.maximum(jnp.dot(x, a1[...], preferred_element_type=f32) + c1[...], 0.0)
  h = jnp.maximum(jnp.dot(h, a2[...], preferred_element_type=f32) + c2[...], 0.0)
  h = jnp.maximum(jnp.dot(h, a3[...], preferred_element_type=f32) + c3[...], 0.0)
  o = jnp.dot(h, a4[...], preferred_element_type=f32) + c4[...]
  o_ref[...] = jnp.sum(o * iemb, axis=1, keepdims=True) * (1.0 / 3.0)


@jax.jit
def kernel(user, item, su_table, ti_table, mlp1, mlp2, mlp3):
  B = user.shape[0]
  V, D = su_table.shape
  uidx = user.astype(jnp.int32)
  iidx = item.astype(jnp.int32)

  pack = _make_pack(V, D)
  main_cols = (V // PBLK) * PBLK
  tail_rows = (V - main_cols) // PACK
  u4_emb, i4_emb = _make_sc_gather(B, PACK * D, main_cols, tail_rows)(
      uidx, iidx, pack(su_table, su_table.T), pack(ti_table, ti_table.T))

  mlps = (mlp1, mlp2, mlp3)
  a1 = jnp.concatenate([m[0][0] for m in mlps], axis=1)
  c1 = jnp.concatenate([m[0][1] for m in mlps])
  H = mlp1[0][0].shape[1]

  def blockdiag(layer):
    z = jnp.zeros((3 * H, 3 * H), jnp.float32)
    for k, m in enumerate(mlps):
      z = z.at[k * H:(k + 1) * H, k * H:(k + 1) * H].set(m[layer][0])
    return z

  a2 = blockdiag(1)
  c2 = jnp.concatenate([m[1][1] for m in mlps])
  a3 = blockdiag(2)
  c3 = jnp.concatenate([m[2][1] for m in mlps])
  a4 = jnp.concatenate([m[3][0] for m in mlps], axis=0)
  c4 = mlp1[3][1] + mlp2[3][1] + mlp3[3][1]

  BLK = 4096
  row_blk = lambda w: pl.BlockSpec((BLK, w), lambda i: (i, 0))
  full = lambda r, c: pl.BlockSpec((r, c), lambda i: (0, 0))
  tail_shift = max(tail_rows.bit_length() - 1, 0)
  score = pl.pallas_call(
      functools.partial(_tc_body, main_cols=main_cols, tail_shift=tail_shift),
      grid=(B // BLK,),
      in_specs=[row_blk(PACK * D), row_blk(PACK * D), row_blk(1), row_blk(1),
                full(D, 3 * H), full(1, 3 * H), full(3 * H, 3 * H),
                full(1, 3 * H), full(3 * H, 3 * H), full(1, 3 * H),
                full(3 * H, D), full(1, D)],
      out_specs=row_blk(1),
      out_shape=jax.ShapeDtypeStruct((B, 1), jnp.float32),
  )(u4_emb, i4_emb, uidx.reshape(B, 1), iidx.reshape(B, 1),
    a1, c1.reshape(1, -1), a2, c2.reshape(1, -1),
    a3, c3.reshape(1, -1), a4, c4.reshape(1, -1))
  return score.reshape(B)
